# Initial kernel scaffold; baseline (speedup 1.0000x reference)
import jax
import jax.numpy as jnp
from jax import lax
from jax.experimental import pallas as pl
from jax.experimental.pallas import tpu as pltpu
from jax.experimental.pallas import tpu_sc as plsc

_N = 10000
_E = 320000
_D = 128
_NPOS = 15
_NC = 2
_NS = 16
_NW = _NC * _NS
_EPW = _E // _NW
_CH = 80
_NCH = _EPW // _CH
_RPS = _N // _NS
_ZR = 125
_NEG = -1e30

_f32 = jnp.float32


def _t1_body(x_ref, vp_ref, pe_ref, wq_ref, wk_ref, hq_ref, hk_ref, hv_ref):
    x = x_ref[...]
    vp = vp_ref[0, 0, :]
    pe = pe_ref[...]
    pos = jnp.zeros_like(x)
    for p in range(_NPOS):
        pos = pos + jnp.where((vp == p)[:, None], pe[p][None, :], 0.0)
    xq = x + pos
    dn = (((1,), (1,)), ((), ()))
    hq_ref[...] = lax.dot_general(xq, wq_ref[...], dn,
                                  precision=lax.Precision.HIGHEST,
                                  preferred_element_type=_f32)
    hk_ref[...] = lax.dot_general(xq, wk_ref[...], dn,
                                  precision=lax.Precision.HIGHEST,
                                  preferred_element_type=_f32)
    hv_ref[...] = lax.dot_general(x, wk_ref[...], dn,
                                  precision=lax.Precision.HIGHEST,
                                  preferred_element_type=_f32)


def _t1(x, visit_pos, pos_emb, w_q, w_k):
    rb = 1000
    grid = (_N // rb,)
    vp3 = visit_pos.reshape(_N // rb, 1, rb)
    return pl.pallas_call(
        _t1_body,
        grid=grid,
        in_specs=[
            pl.BlockSpec((rb, _D), lambda i: (i, 0)),
            pl.BlockSpec((1, 1, rb), lambda i: (i, 0, 0)),
            pl.BlockSpec((_NPOS, _D), lambda i: (0, 0)),
            pl.BlockSpec((_D, _D), lambda i: (0, 0)),
            pl.BlockSpec((_D, _D), lambda i: (0, 0)),
        ],
        out_specs=[
            pl.BlockSpec((rb, _D), lambda i: (i, 0)),
            pl.BlockSpec((rb, _D), lambda i: (i, 0)),
            pl.BlockSpec((rb, _D), lambda i: (i, 0)),
        ],
        out_shape=[jax.ShapeDtypeStruct((_N, _D), _f32)] * 3,
    )(x, vp3, pos_emb, w_q, w_k)


def _k1_body(ei_ref, hq_ref, hk_ref, scores_ref, locmax_ref,
             srcw, dstw, scw, segm, qrows, krows, sem):
    c = lax.axis_index("c")
    s = lax.axis_index("s")
    w = c * _NS + s
    ebase = w * _EPW
    pltpu.sync_copy(ei_ref.at[0, pl.ds(ebase, _EPW)], srcw)
    pltpu.sync_copy(ei_ref.at[1, pl.ds(ebase, _EPW)], dstw)

    def init_body(i, carry):
        segm[pl.ds(i * 16, 16)] = jnp.full((16,), _NEG, _f32)
        return carry
    lax.fori_loop(0, _N // 16, init_body, 0)

    def chunk(i, carry):
        off = i * _CH
        cq = pltpu.async_copy(hq_ref.at[srcw.at[pl.ds(off, _CH)]], qrows, sem)
        cq.wait()
        ck = pltpu.async_copy(hk_ref.at[dstw.at[pl.ds(off, _CH)]], krows, sem)
        ck.wait()

        def edge(j, ecarry):
            acc = jnp.zeros((16,), _f32)
            for k in range(8):
                acc = acc + (qrows[j, pl.ds(k * 16, 16)]
                             * krows[j, pl.ds(k * 16, 16)])
            scw[off + j] = jnp.sum(acc)
            return ecarry
        lax.fori_loop(0, _CH, edge, 0)

        def upd(t, ucarry):
            b = off + t * 16
            d = dstw[pl.ds(b, 16)]
            v = scw[pl.ds(b, 16)]

            def cond(mask):
                return jnp.any(mask)

            def body(mask):
                cur = plsc.load_gather(segm, [d])
                new = jnp.maximum(cur, v)
                plsc.store_scatter(segm, [d], new, mask=mask)
                cur2 = plsc.load_gather(segm, [d])
                return mask & (cur2 < new)

            lax.while_loop(cond, body, jnp.full((16,), True))
            return ucarry
        lax.fori_loop(0, _CH // 16, upd, 0)
        return carry
    lax.fori_loop(0, _NCH, chunk, 0)

    pltpu.sync_copy(scw, scores_ref.at[pl.ds(ebase, _EPW)])
    pltpu.sync_copy(segm, locmax_ref.at[w])


def _k1(edge_index, hq, hk):
    return pl.kernel(
        _k1_body,
        out_type=[
            jax.ShapeDtypeStruct((_E,), _f32),
            jax.ShapeDtypeStruct((_NW, _N), _f32),
        ],
        mesh=plsc.VectorSubcoreMesh(core_axis_name="c", subcore_axis_name="s"),
        scratch_types=[
            pltpu.VMEM((_EPW,), jnp.int32),
            pltpu.VMEM((_EPW,), jnp.int32),
            pltpu.VMEM((_EPW,), _f32),
            pltpu.VMEM((_N,), _f32),
            pltpu.VMEM((_CH, _D), _f32),
            pltpu.VMEM((_CH, _D), _f32),
            pltpu.SemaphoreType.DMA,
        ],
    )(edge_index, hq, hk)


def _t2_body(lm_ref, out_ref):
    m = jnp.max(lm_ref[...], axis=0, keepdims=True)
    out_ref[...] = jnp.where(m <= -1e29, 0.0, m)


def _t2(locmax):
    out = pl.pallas_call(
        _t2_body,
        out_shape=jax.ShapeDtypeStruct((1, _N), _f32),
    )(locmax)
    return out.reshape(_N)


def _k3_body(ei_ref, sc_ref, sm_ref, hv_ref, num_ref, den_ref,
             num_sh, srcw, dstw, scw, smv, denl, vrows, zbuf, exb, dchunk,
             sem):
    c = lax.axis_index("c")
    s = lax.axis_index("s")
    w = c * _NS + s
    ebase = w * _EPW
    pltpu.sync_copy(ei_ref.at[0, pl.ds(ebase, _EPW)], srcw)
    pltpu.sync_copy(ei_ref.at[1, pl.ds(ebase, _EPW)], dstw)
    pltpu.sync_copy(sc_ref.at[pl.ds(ebase, _EPW)], scw)
    pltpu.sync_copy(sm_ref, smv)

    def zero_den(i, carry):
        denl[pl.ds(i * 16, 16)] = jnp.zeros((16,), _f32)
        return carry
    lax.fori_loop(0, _N // 16, zero_den, 0)

    def zero_zb(i, carry):
        r = i // 8
        k = i % 8
        zbuf[r, pl.ds(k * 16, 16)] = jnp.zeros((16,), _f32)
        return carry
    lax.fori_loop(0, _ZR * 8, zero_zb, 0)

    for r in range(_RPS // _ZR):
        pltpu.sync_copy(zbuf, num_sh.at[pl.ds(s * _RPS + r * _ZR, _ZR)])
    plsc.subcore_barrier()

    def chunk(i, carry):
        off = i * _CH
        cv = pltpu.async_copy(hv_ref.at[srcw.at[pl.ds(off, _CH)]], vrows, sem)
        cv.wait()

        def sub(t, scarry):
            b = off + t * 16
            d = dstw[pl.ds(b, 16)]
            sc = scw[pl.ds(b, 16)]
            m = plsc.load_gather(smv, [d])
            ex = jnp.exp(sc - m)
            plsc.addupdate_scatter(denl, [d], ex)
            exb[pl.ds(t * 16, 16)] = ex
            dchunk[pl.ds(t * 16, 16)] = d
            return scarry
        lax.fori_loop(0, _CH // 16, sub, 0)

        def rowscale(j, rcarry):
            e = exb[j]
            for k in range(8):
                vrows[j, pl.ds(k * 16, 16)] = vrows[j, pl.ds(k * 16, 16)] * e
            return rcarry
        lax.fori_loop(0, _CH, rowscale, 0)

        pltpu.sync_copy(vrows, num_sh.at[dchunk], add=True)
        return carry
    lax.fori_loop(0, _NCH, chunk, 0)

    plsc.subcore_barrier()
    for r in range(_RPS // _ZR):
        rows = pl.ds(s * _RPS + r * _ZR, _ZR)
        pltpu.sync_copy(num_sh.at[rows], num_ref.at[c].at[rows])
    pltpu.sync_copy(denl, den_ref.at[w])


def _k3(edge_index, scores, segmax, hv):
    return pl.kernel(
        _k3_body,
        out_type=[
            jax.ShapeDtypeStruct((_NC, _N, _D), _f32),
            jax.ShapeDtypeStruct((_NW, _N), _f32),
        ],
        mesh=plsc.VectorSubcoreMesh(core_axis_name="c", subcore_axis_name="s"),
        scratch_types=[
            pltpu.VMEM_SHARED((_N, _D), _f32),
            pltpu.VMEM((_EPW,), jnp.int32),
            pltpu.VMEM((_EPW,), jnp.int32),
            pltpu.VMEM((_EPW,), _f32),
            pltpu.VMEM((_N,), _f32),
            pltpu.VMEM((_N,), _f32),
            pltpu.VMEM((_CH, _D), _f32),
            pltpu.VMEM((_ZR, _D), _f32),
            pltpu.VMEM((_CH,), _f32),
            pltpu.VMEM((_CH,), jnp.int32),
            pltpu.SemaphoreType.DMA,
        ],
    )(edge_index, scores, segmax, hv)


def _t3_body(num_ref, den_ref, x_ref, wo_ref, lns_ref, lnb_ref, out_ref):
    num = jnp.sum(num_ref[...], axis=0)
    den = jnp.sum(den_ref[...], axis=0)
    agg = num / (den[:, None] + 1e-16)
    mu = jnp.mean(agg, axis=-1, keepdims=True)
    var = jnp.mean((agg - mu) ** 2, axis=-1, keepdims=True)
    normed = (agg - mu) / jnp.sqrt(var + 1e-5)
    normed = normed * lns_ref[0][None, :] + lnb_ref[0][None, :]
    x = x_ref[...]
    out = lax.dot_general(normed, wo_ref[...], (((1,), (1,)), ((), ())),
                          precision=lax.Precision.HIGHEST,
                          preferred_element_type=_f32) + x
    out_ref[...] = jnp.where((den > 0.0)[:, None], out, x)


def _t3(num, dens, x, w_out, ln_scale, ln_bias):
    rb = 1000
    grid = (_N // rb,)
    return pl.pallas_call(
        _t3_body,
        grid=grid,
        in_specs=[
            pl.BlockSpec((_NC, rb, _D), lambda i: (0, i, 0)),
            pl.BlockSpec((_NW, rb), lambda i: (0, i)),
            pl.BlockSpec((rb, _D), lambda i: (i, 0)),
            pl.BlockSpec((_D, _D), lambda i: (0, 0)),
            pl.BlockSpec((1, _D), lambda i: (0, 0)),
            pl.BlockSpec((1, _D), lambda i: (0, 0)),
        ],
        out_specs=pl.BlockSpec((rb, _D), lambda i: (i, 0)),
        out_shape=jax.ShapeDtypeStruct((_N, _D), _f32),
    )(num, dens, x, w_out, ln_scale.reshape(1, _D), ln_bias.reshape(1, _D))


def kernel(x, edge_index, visit_pos, pos_emb, W_q, W_k, W_out,
           ln_scale, ln_bias):
    hq, hk, hv = _t1(x, visit_pos, pos_emb, W_q, W_k)
    scores, locmax = _k1(edge_index, hq, hk)
    segmax = _t2(locmax)
    num, dens = _k3(edge_index, scores, segmax, hv)
    return _t3(num, dens, x, W_out, ln_scale, ln_bias)


# speedup vs baseline: 4.1164x; 4.1164x over previous
import jax
import jax.numpy as jnp
from jax import lax
from jax.experimental import pallas as pl
from jax.experimental.pallas import tpu as pltpu
from jax.experimental.pallas import tpu_sc as plsc

_N = 10000
_E = 320000
_D = 128
_NPOS = 15
_NC = 2
_NS = 16
_NW = _NC * _NS
_EPW = _E // _NW
_CH = 80
_NCH = _EPW // _CH
_RPS = _N // _NS
_SUP = 2000
_ZB = 25
_NEG = -1e30

_f32 = jnp.float32


def _t1_body(x_ref, vp_ref, pe_ref, wq_ref, wk_ref, hq_ref, hk_ref, hv_ref):
    x = x_ref[...]
    vp = vp_ref[...]
    pe = pe_ref[...]
    pe16 = jnp.concatenate([pe, jnp.zeros((1, _D), _f32)], axis=0)
    oh = (vp == lax.broadcasted_iota(jnp.int32, (vp.shape[0], 16), 1))
    dn = (((1,), (1,)), ((), ()))
    pos = lax.dot_general(oh.astype(_f32), pe16, (((1,), (0,)), ((), ())),
                          precision=lax.Precision.HIGHEST,
                          preferred_element_type=_f32)
    xq = x + pos
    hq_ref[...] = lax.dot_general(xq, wq_ref[...], dn,
                                  precision=lax.Precision.HIGHEST,
                                  preferred_element_type=_f32)
    hk_ref[...] = lax.dot_general(xq, wk_ref[...], dn,
                                  precision=lax.Precision.HIGHEST,
                                  preferred_element_type=_f32)
    hv_ref[...] = lax.dot_general(x, wk_ref[...], dn,
                                  precision=lax.Precision.HIGHEST,
                                  preferred_element_type=_f32)


def _t1(x, visit_pos, pos_emb, w_q, w_k):
    rb = 1000
    grid = (_N // rb,)
    vp2 = visit_pos.reshape(_N, 1)
    return pl.pallas_call(
        _t1_body,
        grid=grid,
        in_specs=[
            pl.BlockSpec((rb, _D), lambda i: (i, 0)),
            pl.BlockSpec((rb, 1), lambda i: (i, 0)),
            pl.BlockSpec((_NPOS, _D), lambda i: (0, 0)),
            pl.BlockSpec((_D, _D), lambda i: (0, 0)),
            pl.BlockSpec((_D, _D), lambda i: (0, 0)),
        ],
        out_specs=[
            pl.BlockSpec((rb, _D), lambda i: (i, 0)),
            pl.BlockSpec((rb, _D), lambda i: (i, 0)),
            pl.BlockSpec((rb, _D), lambda i: (i, 0)),
        ],
        out_shape=[jax.ShapeDtypeStruct((_N, _D), _f32)] * 3,
    )(x, vp2, pos_emb, w_q, w_k)


def _k1_body(src_ref, dst_ref, hq_ref, hk_ref, scores_ref, locmax_ref,
             srcw, dstw, scw, segm, qrows, krows, sem):
    c = lax.axis_index("c")
    s = lax.axis_index("s")
    w = c * _NS + s
    ebase = w * _EPW
    pltpu.sync_copy(src_ref.at[pl.ds(ebase, _EPW)], srcw)
    pltpu.sync_copy(dst_ref.at[pl.ds(ebase, _EPW)], dstw)

    def init_body(i, carry):
        segm[pl.ds(i * 16, 16)] = jnp.full((16,), _NEG, _f32)
        return carry
    lax.fori_loop(0, _N // 16, init_body, 0)

    def chunk(i, carry):
        off = i * _CH
        cq = pltpu.async_copy(hq_ref.at[srcw.at[pl.ds(off, _CH)]], qrows, sem)
        cq.wait()
        ck = pltpu.async_copy(hk_ref.at[dstw.at[pl.ds(off, _CH)]], krows, sem)
        ck.wait()

        def grp(t, gcarry):
            b = off + t * 16
            rows = t * 16 + lax.broadcasted_iota(jnp.int32, (16,), 0)
            acc = jnp.zeros((16,), _f32)
            for dcol in range(_D):
                cidx = jnp.full((16,), dcol, jnp.int32)
                qv = plsc.load_gather(qrows, [rows, cidx])
                kv = plsc.load_gather(krows, [rows, cidx])
                acc = acc + qv * kv
            scw[pl.ds(b, 16)] = acc
            d = dstw[pl.ds(b, 16)]

            def cond(mask):
                return jnp.any(mask)

            def body(mask):
                cur = plsc.load_gather(segm, [d])
                new = jnp.maximum(cur, acc)
                plsc.store_scatter(segm, [d], new, mask=mask)
                cur2 = plsc.load_gather(segm, [d])
                return mask & (cur2 < new)

            lax.while_loop(cond, body, jnp.full((16,), True))
            return gcarry
        lax.fori_loop(0, _CH // 16, grp, 0)
        return carry
    lax.fori_loop(0, _NCH, chunk, 0)

    pltpu.sync_copy(scw, scores_ref.at[pl.ds(ebase, _EPW)])
    pltpu.sync_copy(segm, locmax_ref.at[w])


def _k1(src, dst, hq, hk):
    return pl.kernel(
        _k1_body,
        out_type=[
            jax.ShapeDtypeStruct((_E,), _f32),
            jax.ShapeDtypeStruct((_NW, _N), _f32),
        ],
        mesh=plsc.VectorSubcoreMesh(core_axis_name="c", subcore_axis_name="s"),
        compiler_params=pltpu.CompilerParams(
            needs_layout_passes=False, use_tc_tiling_on_sc=False),
        scratch_types=[
            pltpu.VMEM((_EPW,), jnp.int32),
            pltpu.VMEM((_EPW,), jnp.int32),
            pltpu.VMEM((_EPW,), _f32),
            pltpu.VMEM((_N,), _f32),
            pltpu.VMEM((_CH, _D), _f32),
            pltpu.VMEM((_CH, _D), _f32),
            pltpu.SemaphoreType.DMA,
        ],
    )(src, dst, hq, hk)


def _t2_body(lm_ref, out_ref):
    m = jnp.max(lm_ref[...], axis=0, keepdims=True)
    out_ref[...] = jnp.where(m <= -1e29, 0.0, m)


def _t2(locmax):
    out = pl.pallas_call(
        _t2_body,
        out_shape=jax.ShapeDtypeStruct((1, _N), _f32),
    )(locmax)
    return out.reshape(_N)


def _k3_body(src_ref, dst_ref, sc_ref, sm_ref, hv_ref, num_ref, den_ref,
             num_sh, srcb, dstb, scb, smv, denl, vrows, zb, exb, dchunk,
             sem):
    c = lax.axis_index("c")
    s = lax.axis_index("s")
    w = c * _NS + s
    ebase = w * _EPW
    pltpu.sync_copy(sm_ref, smv)

    def zero_den(i, carry):
        denl[pl.ds(i * 16, 16)] = jnp.zeros((16,), _f32)
        return carry
    lax.fori_loop(0, _N // 16, zero_den, 0)

    def zero_zb(i, carry):
        r = i // 8
        k = i % 8
        zb[r, pl.ds(k * 16, 16)] = jnp.zeros((16,), _f32)
        return carry
    lax.fori_loop(0, _ZB * 8, zero_zb, 0)

    for r in range(_RPS // _ZB):
        pltpu.sync_copy(zb, num_sh.at[pl.ds(s * _RPS + r * _ZB, _ZB)])
    plsc.subcore_barrier()

    def sup(u, ucarry):
        sbase = ebase + u * _SUP
        pltpu.sync_copy(src_ref.at[pl.ds(sbase, _SUP)], srcb)
        pltpu.sync_copy(dst_ref.at[pl.ds(sbase, _SUP)], dstb)
        pltpu.sync_copy(sc_ref.at[pl.ds(sbase, _SUP)], scb)

        def chunk(i, carry):
            off = i * _CH
            cv = pltpu.async_copy(hv_ref.at[srcb.at[pl.ds(off, _CH)]],
                                  vrows, sem)
            cv.wait()

            def sub(t, scarry):
                b = off + t * 16
                d = dstb[pl.ds(b, 16)]
                sc = scb[pl.ds(b, 16)]
                m = plsc.load_gather(smv, [d])
                ex = jnp.exp(sc - m)
                plsc.addupdate_scatter(denl, [d], ex)
                exb[pl.ds(t * 16, 16)] = ex
                dchunk[pl.ds(t * 16, 16)] = d
                return scarry
            lax.fori_loop(0, _CH // 16, sub, 0)

            def rowscale(j, rcarry):
                ev = plsc.load_gather(exb, [jnp.full((16,), j, jnp.int32)])
                for k in range(8):
                    vrows[j, pl.ds(k * 16, 16)] = (
                        vrows[j, pl.ds(k * 16, 16)] * ev)
                return rcarry
            lax.fori_loop(0, _CH, rowscale, 0)

            pltpu.sync_copy(vrows, num_sh.at[dchunk], add=True)
            return carry
        lax.fori_loop(0, _SUP // _CH, chunk, 0)
        return ucarry
    lax.fori_loop(0, _EPW // _SUP, sup, 0)

    plsc.subcore_barrier()
    rows = pl.ds(s * _RPS, _RPS)
    pltpu.sync_copy(num_sh.at[rows], num_ref.at[c].at[rows])
    pltpu.sync_copy(denl, den_ref.at[w])


def _k3(src, dst, scores, segmax, hv):
    return pl.kernel(
        _k3_body,
        out_type=[
            jax.ShapeDtypeStruct((_NC, _N, _D), _f32),
            jax.ShapeDtypeStruct((_NW, _N), _f32),
        ],
        mesh=plsc.VectorSubcoreMesh(core_axis_name="c", subcore_axis_name="s"),
        compiler_params=pltpu.CompilerParams(
            needs_layout_passes=False, use_tc_tiling_on_sc=False),
        scratch_types=[
            pltpu.VMEM_SHARED((_N, _D), _f32),
            pltpu.VMEM((_SUP,), jnp.int32),
            pltpu.VMEM((_SUP,), jnp.int32),
            pltpu.VMEM((_SUP,), _f32),
            pltpu.VMEM((_N,), _f32),
            pltpu.VMEM((_N,), _f32),
            pltpu.VMEM((_CH, _D), _f32),
            pltpu.VMEM((_ZB, _D), _f32),
            pltpu.VMEM((_CH,), _f32),
            pltpu.VMEM((_CH,), jnp.int32),
            pltpu.SemaphoreType.DMA,
        ],
    )(src, dst, scores, segmax, hv)


def _t3_body(num_ref, den_ref, x_ref, wo_ref, lns_ref, lnb_ref, out_ref):
    num = jnp.sum(num_ref[...], axis=0)
    den = jnp.sum(den_ref[...], axis=1)
    agg = num / (den[:, None] + 1e-16)
    mu = jnp.mean(agg, axis=-1, keepdims=True)
    var = jnp.mean((agg - mu) ** 2, axis=-1, keepdims=True)
    normed = (agg - mu) / jnp.sqrt(var + 1e-5)
    normed = normed * lns_ref[0][None, :] + lnb_ref[0][None, :]
    x = x_ref[...]
    out = lax.dot_general(normed, wo_ref[...], (((1,), (1,)), ((), ())),
                          precision=lax.Precision.HIGHEST,
                          preferred_element_type=_f32) + x
    out_ref[...] = jnp.where((den > 0.0)[:, None], out, x)


def _t3(num, dens, x, w_out, ln_scale, ln_bias):
    rb = 1000
    grid = (_N // rb,)
    return pl.pallas_call(
        _t3_body,
        grid=grid,
        in_specs=[
            pl.BlockSpec((_NC, rb, _D), lambda i: (0, i, 0)),
            pl.BlockSpec((rb, _NW), lambda i: (i, 0)),
            pl.BlockSpec((rb, _D), lambda i: (i, 0)),
            pl.BlockSpec((_D, _D), lambda i: (0, 0)),
            pl.BlockSpec((1, _D), lambda i: (0, 0)),
            pl.BlockSpec((1, _D), lambda i: (0, 0)),
        ],
        out_specs=pl.BlockSpec((rb, _D), lambda i: (i, 0)),
        out_shape=jax.ShapeDtypeStruct((_N, _D), _f32),
    )(num, dens.T, x, w_out, ln_scale.reshape(1, _D), ln_bias.reshape(1, _D))


def kernel(x, edge_index, visit_pos, pos_emb, W_q, W_k, W_out,
           ln_scale, ln_bias):
    hq, hk, hv = _t1(x, visit_pos, pos_emb, W_q, W_k)
    src = edge_index[0]
    dst = edge_index[1]
    scores, locmax = _k1(src, dst, hq, hk)
    segmax = _t2(locmax)
    num, dens = _k3(src, dst, scores, segmax, hv)
    return _t3(num, dens, x, W_out, ln_scale, ln_bias)


# speedup vs baseline: 4.1397x; 1.0057x over previous
import jax
import jax.numpy as jnp
from jax import lax
from jax.experimental import pallas as pl
from jax.experimental.pallas import tpu as pltpu
from jax.experimental.pallas import tpu_sc as plsc

_N = 10000
_E = 320000
_D = 128
_NPOS = 15
_NC = 2
_NS = 16
_NW = _NC * _NS
_EPW = _E // _NW
_CH = 80
_NCH = _EPW // _CH
_RPS = _N // _NS
_SUP = 2000
_ZB = 25
_NEG = -1e30

_f32 = jnp.float32


def _bf16_split3(a):
    a0 = a.astype(jnp.bfloat16)
    r1 = a - a0.astype(_f32)
    a1 = r1.astype(jnp.bfloat16)
    a2 = (r1 - a1.astype(_f32)).astype(jnp.bfloat16)
    return a0, a1, a2


def _mm_f32(a, b):
    dn = (((1,), (1,)), ((), ()))
    a0, a1, a2 = _bf16_split3(a)
    b0, b1, b2 = _bf16_split3(b)

    def mm(u, v):
        return lax.dot_general(u, v, dn, preferred_element_type=_f32)

    return (((mm(a2, b0) + mm(a1, b1)) + mm(a0, b2))
            + (mm(a1, b0) + mm(a0, b1))) + mm(a0, b0)


def _t1_body(x_ref, vp_ref, pe_ref, wq_ref, wk_ref, hq_ref, hk_ref, hv_ref):
    x = x_ref[...]
    vp = vp_ref[...]
    pe = pe_ref[...]
    pe16 = jnp.concatenate([pe, jnp.zeros((1, _D), _f32)], axis=0)
    oh = (vp == lax.broadcasted_iota(jnp.int32, (vp.shape[0], 16), 1))
    pos = lax.dot_general(oh.astype(_f32), pe16, (((1,), (0,)), ((), ())),
                          precision=lax.Precision.HIGHEST,
                          preferred_element_type=_f32)
    xq = x + pos
    hq_ref[...] = _mm_f32(xq, wq_ref[...])
    hk_ref[...] = _mm_f32(xq, wk_ref[...])
    hv_ref[...] = _mm_f32(x, wk_ref[...])


def _t1(x, visit_pos, pos_emb, w_q, w_k):
    rb = 1000
    grid = (_N // rb,)
    vp2 = visit_pos.reshape(_N, 1)
    return pl.pallas_call(
        _t1_body,
        grid=grid,
        in_specs=[
            pl.BlockSpec((rb, _D), lambda i: (i, 0)),
            pl.BlockSpec((rb, 1), lambda i: (i, 0)),
            pl.BlockSpec((_NPOS, _D), lambda i: (0, 0)),
            pl.BlockSpec((_D, _D), lambda i: (0, 0)),
            pl.BlockSpec((_D, _D), lambda i: (0, 0)),
        ],
        out_specs=[
            pl.BlockSpec((rb, _D), lambda i: (i, 0)),
            pl.BlockSpec((rb, _D), lambda i: (i, 0)),
            pl.BlockSpec((rb, _D), lambda i: (i, 0)),
        ],
        out_shape=[jax.ShapeDtypeStruct((_N, _D), _f32)] * 3,
    )(x, vp2, pos_emb, w_q, w_k)


def _k1_body(src_ref, dst_ref, hq_ref, hk_ref, scores_ref, locmax_ref,
             srcw, dstw, scw, segm, qrows, krows, sem):
    c = lax.axis_index("c")
    s = lax.axis_index("s")
    w = c * _NS + s
    ebase = w * _EPW
    pltpu.sync_copy(src_ref.at[pl.ds(ebase, _EPW)], srcw)
    pltpu.sync_copy(dst_ref.at[pl.ds(ebase, _EPW)], dstw)

    def init_body(i, carry):
        segm[pl.ds(i * 16, 16)] = jnp.full((16,), _NEG, _f32)
        return carry
    lax.fori_loop(0, _N // 16, init_body, 0)

    def chunk(i, carry):
        off = i * _CH
        cq = pltpu.async_copy(hq_ref.at[srcw.at[pl.ds(off, _CH)]], qrows, sem)
        cq.wait()
        ck = pltpu.async_copy(hk_ref.at[dstw.at[pl.ds(off, _CH)]], krows, sem)
        ck.wait()

        def grp(t, gcarry):
            b = off + t * 16
            rows = t * 16 + lax.broadcasted_iota(jnp.int32, (16,), 0)
            acc = jnp.zeros((16,), _f32)
            for dcol in range(_D):
                cidx = jnp.full((16,), dcol, jnp.int32)
                qv = plsc.load_gather(qrows, [rows, cidx])
                kv = plsc.load_gather(krows, [rows, cidx])
                acc = acc + qv * kv
            scw[pl.ds(b, 16)] = acc
            d = dstw[pl.ds(b, 16)]

            def cond(mask):
                return jnp.any(mask)

            def body(mask):
                cur = plsc.load_gather(segm, [d])
                new = jnp.maximum(cur, acc)
                plsc.store_scatter(segm, [d], new, mask=mask)
                cur2 = plsc.load_gather(segm, [d])
                return mask & (cur2 < new)

            lax.while_loop(cond, body, jnp.full((16,), True))
            return gcarry
        lax.fori_loop(0, _CH // 16, grp, 0)
        return carry
    lax.fori_loop(0, _NCH, chunk, 0)

    pltpu.sync_copy(scw, scores_ref.at[pl.ds(ebase, _EPW)])
    pltpu.sync_copy(segm, locmax_ref.at[w])


def _k1(src, dst, hq, hk):
    return pl.kernel(
        _k1_body,
        out_type=[
            jax.ShapeDtypeStruct((_E,), _f32),
            jax.ShapeDtypeStruct((_NW, _N), _f32),
        ],
        mesh=plsc.VectorSubcoreMesh(core_axis_name="c", subcore_axis_name="s"),
        compiler_params=pltpu.CompilerParams(
            needs_layout_passes=False, use_tc_tiling_on_sc=False),
        scratch_types=[
            pltpu.VMEM((_EPW,), jnp.int32),
            pltpu.VMEM((_EPW,), jnp.int32),
            pltpu.VMEM((_EPW,), _f32),
            pltpu.VMEM((_N,), _f32),
            pltpu.VMEM((_CH, _D), _f32),
            pltpu.VMEM((_CH, _D), _f32),
            pltpu.SemaphoreType.DMA,
        ],
    )(src, dst, hq, hk)


def _t2_body(lm_ref, out_ref):
    m = jnp.max(lm_ref[...], axis=0, keepdims=True)
    out_ref[...] = jnp.where(m <= -1e29, 0.0, m)


def _t2(locmax):
    out = pl.pallas_call(
        _t2_body,
        out_shape=jax.ShapeDtypeStruct((1, _N), _f32),
    )(locmax)
    return out.reshape(_N)


def _k3_body(src_ref, dst_ref, sc_ref, sm_ref, hv_ref, num_ref, den_ref,
             num_sh, srcb, dstb, scb, smv, denl, vrows, zb, exb, dchunk,
             sem):
    c = lax.axis_index("c")
    s = lax.axis_index("s")
    w = c * _NS + s
    ebase = w * _EPW
    pltpu.sync_copy(sm_ref, smv)

    def zero_den(i, carry):
        denl[pl.ds(i * 16, 16)] = jnp.zeros((16,), _f32)
        return carry
    lax.fori_loop(0, _N // 16, zero_den, 0)

    def zero_zb(i, carry):
        r = i // 8
        k = i % 8
        zb[r, pl.ds(k * 16, 16)] = jnp.zeros((16,), _f32)
        return carry
    lax.fori_loop(0, _ZB * 8, zero_zb, 0)

    for r in range(_RPS // _ZB):
        pltpu.sync_copy(zb, num_sh.at[pl.ds(s * _RPS + r * _ZB, _ZB)])
    plsc.subcore_barrier()

    def sup(u, ucarry):
        sbase = ebase + u * _SUP
        pltpu.sync_copy(src_ref.at[pl.ds(sbase, _SUP)], srcb)
        pltpu.sync_copy(dst_ref.at[pl.ds(sbase, _SUP)], dstb)
        pltpu.sync_copy(sc_ref.at[pl.ds(sbase, _SUP)], scb)

        def chunk(i, carry):
            off = i * _CH
            cv = pltpu.async_copy(hv_ref.at[srcb.at[pl.ds(off, _CH)]],
                                  vrows, sem)
            cv.wait()

            def sub(t, scarry):
                b = off + t * 16
                d = dstb[pl.ds(b, 16)]
                sc = scb[pl.ds(b, 16)]
                m = plsc.load_gather(smv, [d])
                ex = jnp.exp(sc - m)
                plsc.addupdate_scatter(denl, [d], ex)
                exb[pl.ds(t * 16, 16)] = ex
                dchunk[pl.ds(t * 16, 16)] = d
                return scarry
            lax.fori_loop(0, _CH // 16, sub, 0)

            def rowscale(j, rcarry):
                ev = plsc.load_gather(exb, [jnp.full((16,), j, jnp.int32)])
                for k in range(8):
                    vrows[j, pl.ds(k * 16, 16)] = (
                        vrows[j, pl.ds(k * 16, 16)] * ev)
                return rcarry
            lax.fori_loop(0, _CH, rowscale, 0)

            pltpu.sync_copy(vrows, num_sh.at[dchunk], add=True)
            return carry
        lax.fori_loop(0, _SUP // _CH, chunk, 0)
        return ucarry
    lax.fori_loop(0, _EPW // _SUP, sup, 0)

    plsc.subcore_barrier()
    rows = pl.ds(s * _RPS, _RPS)
    pltpu.sync_copy(num_sh.at[rows], num_ref.at[c].at[rows])
    pltpu.sync_copy(denl, den_ref.at[w])


def _k3(src, dst, scores, segmax, hv):
    return pl.kernel(
        _k3_body,
        out_type=[
            jax.ShapeDtypeStruct((_NC, _N, _D), _f32),
            jax.ShapeDtypeStruct((_NW, _N), _f32),
        ],
        mesh=plsc.VectorSubcoreMesh(core_axis_name="c", subcore_axis_name="s"),
        compiler_params=pltpu.CompilerParams(
            needs_layout_passes=False, use_tc_tiling_on_sc=False),
        scratch_types=[
            pltpu.VMEM_SHARED((_N, _D), _f32),
            pltpu.VMEM((_SUP,), jnp.int32),
            pltpu.VMEM((_SUP,), jnp.int32),
            pltpu.VMEM((_SUP,), _f32),
            pltpu.VMEM((_N,), _f32),
            pltpu.VMEM((_N,), _f32),
            pltpu.VMEM((_CH, _D), _f32),
            pltpu.VMEM((_ZB, _D), _f32),
            pltpu.VMEM((_CH,), _f32),
            pltpu.VMEM((_CH,), jnp.int32),
            pltpu.SemaphoreType.DMA,
        ],
    )(src, dst, scores, segmax, hv)


def _t3_body(num_ref, den_ref, x_ref, wo_ref, lns_ref, lnb_ref, out_ref):
    num = jnp.sum(num_ref[...], axis=0)
    den = jnp.sum(den_ref[...], axis=1)
    agg = num / (den[:, None] + 1e-16)
    mu = jnp.mean(agg, axis=-1, keepdims=True)
    var = jnp.mean((agg - mu) ** 2, axis=-1, keepdims=True)
    normed = (agg - mu) / jnp.sqrt(var + 1e-5)
    normed = normed * lns_ref[0][None, :] + lnb_ref[0][None, :]
    x = x_ref[...]
    out = lax.dot_general(normed, wo_ref[...], (((1,), (1,)), ((), ())),
                          precision=lax.Precision.HIGHEST,
                          preferred_element_type=_f32) + x
    out_ref[...] = jnp.where((den > 0.0)[:, None], out, x)


def _t3(num, dens, x, w_out, ln_scale, ln_bias):
    rb = 1000
    grid = (_N // rb,)
    return pl.pallas_call(
        _t3_body,
        grid=grid,
        in_specs=[
            pl.BlockSpec((_NC, rb, _D), lambda i: (0, i, 0)),
            pl.BlockSpec((rb, _NW), lambda i: (i, 0)),
            pl.BlockSpec((rb, _D), lambda i: (i, 0)),
            pl.BlockSpec((_D, _D), lambda i: (0, 0)),
            pl.BlockSpec((1, _D), lambda i: (0, 0)),
            pl.BlockSpec((1, _D), lambda i: (0, 0)),
        ],
        out_specs=pl.BlockSpec((rb, _D), lambda i: (i, 0)),
        out_shape=jax.ShapeDtypeStruct((_N, _D), _f32),
    )(num, dens.T, x, w_out, ln_scale.reshape(1, _D), ln_bias.reshape(1, _D))


def kernel(x, edge_index, visit_pos, pos_emb, W_q, W_k, W_out,
           ln_scale, ln_bias):
    hq, hk, hv = _t1(x, visit_pos, pos_emb, W_q, W_k)
    src = edge_index[0]
    dst = edge_index[1]
    scores, locmax = _k1(src, dst, hq, hk)
    segmax = _t2(locmax)
    num, dens = _k3(src, dst, scores, segmax, hv)
    return _t3(num, dens, x, W_out, ln_scale, ln_bias)
